# baseline (speedup 1.0000x reference)
import functools

import jax
import jax.numpy as jnp
from jax import lax
from jax.experimental import pallas as pl
from jax.experimental.pallas import tpu as pltpu
from jax.experimental.pallas import tpu_sc as plsc

N_USERS = 25000
N_ITEMS = 25000
D = 64
E = 800000
K = 4
ALPHA_HALF = 0.5

NT = 32
R_TILE = 1568
HALF = 25088
N_PAD = 2 * HALF
PADROWS = HALF - N_USERS
CHUNK = 128
E_TILE = 25088
E_PAD = NT * E_TILE
E_SC = E_PAD // 2

_SC_MESH = plsc.VectorSubcoreMesh(core_axis_name="c", subcore_axis_name="s")
_IOTA16 = None


def _bucket_of(rowv):
    return ((rowv >> 5) * 669) >> 15


def _dup_stats(bv, iota):
    one = jnp.ones((16,), jnp.int32)
    zero = jnp.zeros((16,), jnp.int32)
    rankv = zero
    countv = zero
    for j in range(16):
        eq = bv == bv[j]
        countv = countv + jnp.where(eq, one, zero)
        rankv = rankv + jnp.where(jnp.logical_and(eq, iota > j), one, zero)
    return rankv, countv


@functools.partial(
    pl.kernel,
    mesh=_SC_MESH,
    compiler_params=pltpu.CompilerParams(use_tc_tiling_on_sc=False,
                                         needs_layout_passes=False),
    out_type=(
        jax.ShapeDtypeStruct((E_PAD,), jnp.int32),
        jax.ShapeDtypeStruct((E_PAD,), jnp.int32),
        jax.ShapeDtypeStruct((E_PAD,), jnp.float32),
        jax.ShapeDtypeStruct((2, 48), jnp.int32),
    ),
    scratch_types=[
        pltpu.VMEM((CHUNK,), jnp.int32),
        pltpu.VMEM((CHUNK,), jnp.int32),
        pltpu.VMEM((CHUNK,), jnp.float32),
        pltpu.VMEM((CHUNK,), jnp.int32),
        pltpu.VMEM((16, 32), jnp.int32),
        pltpu.VMEM((32,), jnp.int32),
        pltpu.VMEM((48,), jnp.int32),
        pltpu.VMEM((32,), jnp.int32),
        pltpu.VMEM_SHARED((16, 32), jnp.int32),
        pltpu.VMEM_SHARED((E_SC,), jnp.int32),
        pltpu.VMEM_SHARED((E_SC,), jnp.int32),
        pltpu.VMEM_SHARED((E_SC,), jnp.float32),
        pltpu.SemaphoreType.DMA,
        pltpu.SemaphoreType.DMA,
    ],
)
def _bucketize(row_hbm, col_hbm, w_hbm,
               rows_out, cols_out, ws_out, bnd_out,
               rowbuf, colbuf, wbuf, dstbuf, cntbuf, mycnt, bndbuf, offs,
               hist_sp, rows_sp, cols_sp, ws_sp, sem_a, sem_b):
    cid = lax.axis_index("c")
    sid = lax.axis_index("s")
    ebase = (cid * 16 + sid) * E_TILE
    iota = lax.iota(jnp.int32, 16)
    zero16 = jnp.zeros((16,), jnp.int32)

    mycnt[pl.ds(0, 16)] = zero16
    mycnt[pl.ds(16, 16)] = zero16

    def _hist_chunk(c, carry):
        pltpu.sync_copy(row_hbm.at[pl.ds(ebase + c * CHUNK, CHUNK)], rowbuf)

        def _grp(g, gc):
            bv = _bucket_of(rowbuf[pl.ds(g * 16, 16)])
            rankv, countv = _dup_stats(bv, iota)
            islast = rankv == countv - 1
            plsc.addupdate_scatter(mycnt, [bv], countv, mask=islast)
            return gc

        return lax.fori_loop(0, CHUNK // 16, _grp, carry)

    lax.fori_loop(0, E_TILE // CHUNK, _hist_chunk, 0)
    pltpu.sync_copy(mycnt, hist_sp.at[sid])
    plsc.subcore_barrier()

    pltpu.sync_copy(hist_sp, cntbuf)
    t0 = zero16
    t1 = zero16
    p0 = zero16
    p1 = zero16
    for t in range(16):
        r0 = cntbuf[t, pl.ds(0, 16)]
        r1 = cntbuf[t, pl.ds(16, 16)]
        m = t < sid
        p0 = p0 + jnp.where(m, r0, 0)
        p1 = p1 + jnp.where(m, r1, 0)
        t0 = t0 + r0
        t1 = t1 + r1
    run = jnp.int32(0)
    ex0 = zero16
    ex1 = zero16
    for b in range(16):
        ex0 = jnp.where(iota == b, run, ex0)
        run = run + t0[b]
    for b in range(16):
        ex1 = jnp.where(iota == b, run, ex1)
        run = run + t1[b]
    offs[pl.ds(0, 16)] = ex0 + p0
    offs[pl.ds(16, 16)] = ex1 + p1

    @pl.when(sid == 0)
    def _write_bnd():
        bndbuf[pl.ds(0, 16)] = ex0
        bndbuf[pl.ds(16, 16)] = ex1
        bndbuf[pl.ds(32, 16)] = jnp.where(iota == 0, E_SC, 0)
        pltpu.sync_copy(bndbuf, bnd_out.at[cid])

    def _scat_chunk(c, carry):
        eb = ebase + c * CHUNK
        d1 = pltpu.async_copy(row_hbm.at[pl.ds(eb, CHUNK)], rowbuf, sem_a)
        d2 = pltpu.async_copy(col_hbm.at[pl.ds(eb, CHUNK)], colbuf, sem_a)
        d3 = pltpu.async_copy(w_hbm.at[pl.ds(eb, CHUNK)], wbuf, sem_a)
        d1.wait()
        d2.wait()
        d3.wait()

        def _grp(g, gc):
            bv = _bucket_of(rowbuf[pl.ds(g * 16, 16)])
            rankv, countv = _dup_stats(bv, iota)
            islast = rankv == countv - 1
            dstv = plsc.load_gather(offs, [bv]) + rankv
            dstbuf[pl.ds(g * 16, 16)] = dstv
            plsc.addupdate_scatter(offs, [bv], countv, mask=islast)
            return gc

        lax.fori_loop(0, CHUNK // 16, _grp, 0)
        pltpu.sync_copy(rowbuf, rows_sp.at[dstbuf])
        pltpu.sync_copy(colbuf, cols_sp.at[dstbuf])
        pltpu.sync_copy(wbuf, ws_sp.at[dstbuf])
        return carry

    lax.fori_loop(0, E_TILE // CHUNK, _scat_chunk, 0)
    plsc.subcore_barrier()

    sl = E_SC // 16
    hb = cid * E_SC + sid * sl
    pltpu.sync_copy(rows_sp.at[pl.ds(sid * sl, sl)], rows_out.at[pl.ds(hb, sl)])
    pltpu.sync_copy(cols_sp.at[pl.ds(sid * sl, sl)], cols_out.at[pl.ds(hb, sl)])
    pltpu.sync_copy(ws_sp.at[pl.ds(sid * sl, sl)], ws_out.at[pl.ds(hb, sl)])


@functools.partial(
    pl.kernel,
    mesh=_SC_MESH,
    compiler_params=pltpu.CompilerParams(use_tc_tiling_on_sc=False,
                                         needs_layout_passes=False),
    out_type=jax.ShapeDtypeStruct((N_PAD, D), jnp.float32),
    scratch_types=[
        pltpu.VMEM((2, CHUNK), jnp.int32),
        pltpu.VMEM((2, CHUNK), jnp.int32),
        pltpu.VMEM((2, CHUNK), jnp.float32),
        pltpu.VMEM((2, CHUNK, D), jnp.float32),
        pltpu.VMEM((96,), jnp.int32),
        pltpu.VMEM((R_TILE, D), jnp.float32),
        pltpu.SemaphoreType.DMA,
        pltpu.SemaphoreType.DMA,
    ],
)
def _layer(x_hbm, cs_hbm, rs_hbm, ws_hbm, bnd_hbm, out_hbm,
           colbuf, rowbuf, wbuf, gbuf, bndbuf, acc, sem_e, sem_g):
    wid = lax.axis_index("s") * 2 + lax.axis_index("c")
    base = wid * R_TILE
    pltpu.sync_copy(bnd_hbm, bndbuf)

    zero16 = jnp.zeros((16,), jnp.float32)

    def _zero_row(r, carry):
        for d4 in range(4):
            acc[r, pl.ds(d4 * 16, 16)] = zero16
        return carry

    lax.fori_loop(0, R_TILE, _zero_row, 0)

    def _issue_edges(c, p):
        eb = c * CHUNK
        pltpu.async_copy(cs_hbm.at[pl.ds(eb, CHUNK)], colbuf.at[p], sem_e)
        pltpu.async_copy(rs_hbm.at[pl.ds(eb, CHUNK)], rowbuf.at[p], sem_e)
        pltpu.async_copy(ws_hbm.at[pl.ds(eb, CHUNK)], wbuf.at[p], sem_e)

    def _drain_edges(c, p):
        eb = c * CHUNK
        pltpu.make_async_copy(cs_hbm.at[pl.ds(eb, CHUNK)], colbuf.at[p], sem_e).wait()
        pltpu.make_async_copy(rs_hbm.at[pl.ds(eb, CHUNK)], rowbuf.at[p], sem_e).wait()
        pltpu.make_async_copy(ws_hbm.at[pl.ds(eb, CHUNK)], wbuf.at[p], sem_e).wait()

    def _issue_gather(p):
        pltpu.async_copy(x_hbm.at[colbuf.at[p]], gbuf.at[p], sem_g)

    def _drain_gather(p):
        pltpu.make_async_copy(x_hbm.at[colbuf.at[p]], gbuf.at[p], sem_g).wait()

    for seg in range(2):
        sc_base = seg * E_SC
        bv = bndbuf[pl.ds(seg * 48 + wid, 16)]
        e0 = bv[0] + sc_base
        e1 = bv[1] + sc_base
        c0 = e0 // CHUNK
        c1 = (e1 + (CHUNK - 1)) // CHUNK

        @pl.when(c0 < c1)
        def _prologue():
            _issue_edges(c0, 0)
            _drain_edges(c0, 0)
            _issue_gather(0)

        def _chunk(c, carry):
            p = lax.rem(c - c0, 2)
            q = 1 - p
            _drain_gather(p)

            @pl.when(c + 1 < c1)
            def _prefetch_edges():
                _issue_edges(c + 1, q)

            iota = lax.iota(jnp.int32, 16)
            pv = jnp.full((16,), p, jnp.int32)

            def _group(g, gcarry):
                rowv = rowbuf[p, pl.ds(g * 16, 16)] - base
                validv = jnp.logical_and(rowv >= 0, rowv < R_TILE)
                wv = jnp.where(validv, wbuf[p, pl.ds(g * 16, 16)], 0.0)
                rlv = jnp.where(validv, rowv, 0)
                ev = g * 16 + iota
                dv = jnp.zeros((16,), jnp.int32)
                for _ in range(D):
                    gv = plsc.load_gather(gbuf, [pv, ev, dv])
                    plsc.addupdate_scatter(acc, [rlv, dv], gv * wv)
                    dv = dv + 1
                return gcarry

            lax.fori_loop(0, CHUNK // 16, _group, 0)

            @pl.when(c + 1 < c1)
            def _prefetch_gather():
                _drain_edges(c + 1, q)
                _issue_gather(q)

            return carry

        lax.fori_loop(c0, c1, _chunk, 0)

    pltpu.sync_copy(acc, out_hbm.at[pl.ds(base, R_TILE)])


def _fusion_body(emb, wu, bu, wi, bi, uc, ic, out):
    b = pl.program_id(0)
    x = emb[...]
    isu = b < 16
    W = jnp.where(isu, wu[...], wi[...])
    bb = jnp.where(isu, bu[...], bi[...])
    C = jnp.where(isu, uc[...], ic[...])
    logits = jnp.dot(x, W, preferred_element_type=jnp.float32) + bb
    m = jnp.max(logits, axis=-1, keepdims=True)
    ex = jnp.exp(logits - m)
    attn = ex / jnp.sum(ex, axis=-1, keepdims=True)
    out[...] = x + ALPHA_HALF * jnp.dot(attn, C, preferred_element_type=jnp.float32)


_fusion = pl.pallas_call(
    _fusion_body,
    grid=(NT,),
    in_specs=[
        pl.BlockSpec((R_TILE, D), lambda b: (b, 0)),
        pl.BlockSpec((D, K), lambda b: (0, 0)),
        pl.BlockSpec((1, K), lambda b: (0, 0)),
        pl.BlockSpec((D, K), lambda b: (0, 0)),
        pl.BlockSpec((1, K), lambda b: (0, 0)),
        pl.BlockSpec((K, D), lambda b: (0, 0)),
        pl.BlockSpec((K, D), lambda b: (0, 0)),
    ],
    out_specs=pl.BlockSpec((R_TILE, D), lambda b: (b, 0)),
    out_shape=jax.ShapeDtypeStruct((N_PAD, D), jnp.float32),
)


def _mean_body(a, b, c, d, out):
    out[...] = 0.25 * (a[...] + b[...] + c[...] + d[...])


_mean = pl.pallas_call(
    _mean_body,
    grid=(NT,),
    in_specs=[pl.BlockSpec((R_TILE, D), lambda b: (b, 0)) for _ in range(4)],
    out_specs=pl.BlockSpec((R_TILE, D), lambda b: (b, 0)),
    out_shape=jax.ShapeDtypeStruct((N_PAD, D), jnp.float32),
)


def kernel(edge_index, edge_weight, user_emb, item_emb, user_coll, item_coll, Wu, bu, Wi, bi):
    row = edge_index[0].astype(jnp.int32)
    col = edge_index[1].astype(jnp.int32)
    row = row + PADROWS * (row >= N_USERS).astype(jnp.int32)
    col = col + PADROWS * (col >= N_USERS).astype(jnp.int32)
    npad = E_PAD - E
    row = jnp.concatenate([row, jnp.full((npad,), N_PAD - 1, jnp.int32)])
    col = jnp.concatenate([col, jnp.zeros((npad,), jnp.int32)])
    w = jnp.concatenate([edge_weight, jnp.zeros((npad,), jnp.float32)])

    rows_g, cols_g, ws_g, bnd = _bucketize(row, col, w)
    bnd = bnd.reshape(96)

    emb_pad = jnp.concatenate(
        [user_emb, jnp.zeros((PADROWS, D), jnp.float32),
         item_emb, jnp.zeros((PADROWS, D), jnp.float32)], axis=0)

    x0 = _fusion(emb_pad, Wu, bu.reshape(1, K), Wi, bi.reshape(1, K),
                 user_coll, item_coll)
    x1 = _layer(x0, cols_g, rows_g, ws_g, bnd)
    x2 = _layer(x1, cols_g, rows_g, ws_g, bnd)
    x3 = _layer(x2, cols_g, rows_g, ws_g, bnd)
    fin = _mean(x0, x1, x2, x3)
    return fin[:N_USERS], fin[HALF:HALF + N_ITEMS]


# speedup vs baseline: 2.9415x; 2.9415x over previous
import functools

import jax
import jax.numpy as jnp
from jax import lax
from jax.experimental import pallas as pl
from jax.experimental.pallas import tpu as pltpu
from jax.experimental.pallas import tpu_sc as plsc

N_USERS = 25000
N_ITEMS = 25000
D = 64
E = 800000
K = 4
ALPHA_HALF = 0.5

NT = 32
R_TILE = 1568
HALF = 25088
N_PAD = 2 * HALF
PADROWS = HALF - N_USERS
CHUNK = 128
E_TILE = 25088
E_PAD = NT * E_TILE
E_SC = E_PAD // 2

_SC_MESH = plsc.VectorSubcoreMesh(core_axis_name="c", subcore_axis_name="s")
_IOTA16 = None


def _bucket_of(rowv):
    return ((rowv >> 5) * 669) >> 15


def _dup_stats(bv, iota):
    one = jnp.ones((16,), jnp.int32)
    zero = jnp.zeros((16,), jnp.int32)
    rankv = zero
    countv = zero
    for j in range(16):
        eq = bv == bv[j]
        countv = countv + jnp.where(eq, one, zero)
        rankv = rankv + jnp.where(jnp.logical_and(eq, iota > j), one, zero)
    return rankv, countv


@functools.partial(
    pl.kernel,
    mesh=_SC_MESH,
    compiler_params=pltpu.CompilerParams(use_tc_tiling_on_sc=False,
                                         needs_layout_passes=False),
    out_type=(
        jax.ShapeDtypeStruct((E_PAD,), jnp.int32),
        jax.ShapeDtypeStruct((E_PAD,), jnp.int32),
        jax.ShapeDtypeStruct((E_PAD,), jnp.float32),
        jax.ShapeDtypeStruct((2, 48), jnp.int32),
    ),
    scratch_types=[
        pltpu.VMEM((CHUNK,), jnp.int32),
        pltpu.VMEM((CHUNK,), jnp.int32),
        pltpu.VMEM((CHUNK,), jnp.float32),
        pltpu.VMEM((CHUNK,), jnp.int32),
        pltpu.VMEM((16, 32), jnp.int32),
        pltpu.VMEM((32,), jnp.int32),
        pltpu.VMEM((48,), jnp.int32),
        pltpu.VMEM((32,), jnp.int32),
        pltpu.VMEM_SHARED((16, 32), jnp.int32),
        pltpu.VMEM_SHARED((E_SC,), jnp.int32),
        pltpu.VMEM_SHARED((E_SC,), jnp.int32),
        pltpu.VMEM_SHARED((E_SC,), jnp.float32),
        pltpu.SemaphoreType.DMA,
        pltpu.SemaphoreType.DMA,
    ],
)
def _bucketize(row_hbm, col_hbm, w_hbm,
               rows_out, cols_out, ws_out, bnd_out,
               rowbuf, colbuf, wbuf, dstbuf, cntbuf, mycnt, bndbuf, offs,
               hist_sp, rows_sp, cols_sp, ws_sp, sem_a, sem_b):
    cid = lax.axis_index("c")
    sid = lax.axis_index("s")
    ebase = (cid * 16 + sid) * E_TILE
    iota = lax.iota(jnp.int32, 16)
    zero16 = jnp.zeros((16,), jnp.int32)

    mycnt[pl.ds(0, 16)] = zero16
    mycnt[pl.ds(16, 16)] = zero16

    def _hist_chunk(c, carry):
        pltpu.sync_copy(row_hbm.at[pl.ds(ebase + c * CHUNK, CHUNK)], rowbuf)

        def _grp(g, gc):
            bv = _bucket_of(rowbuf[pl.ds(g * 16, 16)])
            rankv, countv = _dup_stats(bv, iota)
            islast = rankv == countv - 1
            plsc.addupdate_scatter(mycnt, [bv], countv, mask=islast)
            return gc

        return lax.fori_loop(0, CHUNK // 16, _grp, carry)

    lax.fori_loop(0, E_TILE // CHUNK, _hist_chunk, 0)
    pltpu.sync_copy(mycnt, hist_sp.at[sid])
    plsc.subcore_barrier()

    pltpu.sync_copy(hist_sp, cntbuf)
    t0 = zero16
    t1 = zero16
    p0 = zero16
    p1 = zero16
    for t in range(16):
        r0 = cntbuf[t, pl.ds(0, 16)]
        r1 = cntbuf[t, pl.ds(16, 16)]
        m = t < sid
        p0 = p0 + jnp.where(m, r0, 0)
        p1 = p1 + jnp.where(m, r1, 0)
        t0 = t0 + r0
        t1 = t1 + r1
    run = jnp.int32(0)
    ex0 = zero16
    ex1 = zero16
    for b in range(16):
        ex0 = jnp.where(iota == b, run, ex0)
        run = run + t0[b]
    for b in range(16):
        ex1 = jnp.where(iota == b, run, ex1)
        run = run + t1[b]
    offs[pl.ds(0, 16)] = ex0 + p0
    offs[pl.ds(16, 16)] = ex1 + p1

    @pl.when(sid == 0)
    def _write_bnd():
        bndbuf[pl.ds(0, 16)] = ex0
        bndbuf[pl.ds(16, 16)] = ex1
        bndbuf[pl.ds(32, 16)] = jnp.where(iota == 0, E_SC, 0)
        pltpu.sync_copy(bndbuf, bnd_out.at[cid])

    def _scat_chunk(c, carry):
        eb = ebase + c * CHUNK
        d1 = pltpu.async_copy(row_hbm.at[pl.ds(eb, CHUNK)], rowbuf, sem_a)
        d2 = pltpu.async_copy(col_hbm.at[pl.ds(eb, CHUNK)], colbuf, sem_a)
        d3 = pltpu.async_copy(w_hbm.at[pl.ds(eb, CHUNK)], wbuf, sem_a)
        d1.wait()
        d2.wait()
        d3.wait()

        def _grp(g, gc):
            bv = _bucket_of(rowbuf[pl.ds(g * 16, 16)])
            rankv, countv = _dup_stats(bv, iota)
            islast = rankv == countv - 1
            dstv = plsc.load_gather(offs, [bv]) + rankv
            dstbuf[pl.ds(g * 16, 16)] = dstv
            plsc.addupdate_scatter(offs, [bv], countv, mask=islast)
            return gc

        lax.fori_loop(0, CHUNK // 16, _grp, 0)
        pltpu.sync_copy(rowbuf, rows_sp.at[dstbuf])
        pltpu.sync_copy(colbuf, cols_sp.at[dstbuf])
        pltpu.sync_copy(wbuf, ws_sp.at[dstbuf])
        return carry

    lax.fori_loop(0, E_TILE // CHUNK, _scat_chunk, 0)
    plsc.subcore_barrier()

    sl = E_SC // 16
    hb = cid * E_SC + sid * sl
    pltpu.sync_copy(rows_sp.at[pl.ds(sid * sl, sl)], rows_out.at[pl.ds(hb, sl)])
    pltpu.sync_copy(cols_sp.at[pl.ds(sid * sl, sl)], cols_out.at[pl.ds(hb, sl)])
    pltpu.sync_copy(ws_sp.at[pl.ds(sid * sl, sl)], ws_out.at[pl.ds(hb, sl)])


@functools.partial(
    pl.kernel,
    mesh=_SC_MESH,
    compiler_params=pltpu.CompilerParams(use_tc_tiling_on_sc=False,
                                         needs_layout_passes=False),
    out_type=jax.ShapeDtypeStruct((N_PAD, D), jnp.float32),
    scratch_types=[
        pltpu.VMEM((2, CHUNK), jnp.int32),
        pltpu.VMEM((2, CHUNK), jnp.int32),
        pltpu.VMEM((2, CHUNK), jnp.float32),
        pltpu.VMEM((2, CHUNK, D), jnp.float32),
        pltpu.VMEM((96,), jnp.int32),
        pltpu.VMEM((R_TILE, D), jnp.float32),
        pltpu.SemaphoreType.DMA,
        pltpu.SemaphoreType.DMA,
    ],
)
def _layer(x_hbm, cs_hbm, rs_hbm, ws_hbm, bnd_hbm, out_hbm,
           colbuf, rowbuf, wbuf, gbuf, bndbuf, acc, sem_e, sem_g):
    wid = lax.axis_index("s") * 2 + lax.axis_index("c")
    base = wid * R_TILE
    pltpu.sync_copy(bnd_hbm, bndbuf)

    zero16 = jnp.zeros((16,), jnp.float32)

    def _zero_row(r, carry):
        for d4 in range(4):
            acc[r, pl.ds(d4 * 16, 16)] = zero16
        return carry

    lax.fori_loop(0, R_TILE, _zero_row, 0)

    def _issue_edges(c, p):
        eb = c * CHUNK
        pltpu.async_copy(cs_hbm.at[pl.ds(eb, CHUNK)], colbuf.at[p], sem_e)
        pltpu.async_copy(rs_hbm.at[pl.ds(eb, CHUNK)], rowbuf.at[p], sem_e)
        pltpu.async_copy(ws_hbm.at[pl.ds(eb, CHUNK)], wbuf.at[p], sem_e)

    def _drain_edges(c, p):
        eb = c * CHUNK
        pltpu.make_async_copy(cs_hbm.at[pl.ds(eb, CHUNK)], colbuf.at[p], sem_e).wait()
        pltpu.make_async_copy(rs_hbm.at[pl.ds(eb, CHUNK)], rowbuf.at[p], sem_e).wait()
        pltpu.make_async_copy(ws_hbm.at[pl.ds(eb, CHUNK)], wbuf.at[p], sem_e).wait()

    def _issue_gather(p):
        pltpu.async_copy(x_hbm.at[colbuf.at[p]], gbuf.at[p], sem_g)

    def _drain_gather(p):
        pltpu.make_async_copy(x_hbm.at[colbuf.at[p]], gbuf.at[p], sem_g).wait()

    for seg in range(2):
        sc_base = seg * E_SC
        bv = bndbuf[pl.ds(seg * 48 + wid, 16)]
        e0 = bv[0] + sc_base
        e1 = bv[1] + sc_base
        c0 = e0 // CHUNK
        c1 = (e1 + (CHUNK - 1)) // CHUNK

        @pl.when(c0 < c1)
        def _prologue():
            _issue_edges(c0, 0)
            _drain_edges(c0, 0)
            _issue_gather(0)

        def _chunk(c, carry):
            p = lax.rem(c - c0, 2)
            q = 1 - p
            _drain_gather(p)

            @pl.when(c + 1 < c1)
            def _prefetch_edges():
                _issue_edges(c + 1, q)

            def _group(g, gcarry):
                rowv = rowbuf[p, pl.ds(g * 16, 16)] - base
                validv = jnp.logical_and(rowv >= 0, rowv < R_TILE)
                wv = jnp.where(validv, wbuf[p, pl.ds(g * 16, 16)], 0.0)
                rlv = jnp.where(validv, rowv, 0)
                ws = [wv[k] for k in range(16)]
                rls = [rlv[k] for k in range(16)]
                for k in range(16):
                    e = g * 16 + k
                    w = ws[k]
                    rl = rls[k]
                    for d4 in range(4):
                        gvec = gbuf[p, e, pl.ds(d4 * 16, 16)]
                        plsc.addupdate(acc.at[rl, pl.ds(d4 * 16, 16)], gvec * w)
                return gcarry

            lax.fori_loop(0, CHUNK // 16, _group, 0)

            @pl.when(c + 1 < c1)
            def _prefetch_gather():
                _drain_edges(c + 1, q)
                _issue_gather(q)

            return carry

        lax.fori_loop(c0, c1, _chunk, 0)

    pltpu.sync_copy(acc, out_hbm.at[pl.ds(base, R_TILE)])


def _fusion_body(emb, wu, bu, wi, bi, uc, ic, out):
    b = pl.program_id(0)
    x = emb[...]
    isu = b < 16
    W = jnp.where(isu, wu[...], wi[...])
    bb = jnp.where(isu, bu[...], bi[...])
    C = jnp.where(isu, uc[...], ic[...])
    logits = jnp.dot(x, W, preferred_element_type=jnp.float32) + bb
    m = jnp.max(logits, axis=-1, keepdims=True)
    ex = jnp.exp(logits - m)
    attn = ex / jnp.sum(ex, axis=-1, keepdims=True)
    out[...] = x + ALPHA_HALF * jnp.dot(attn, C, preferred_element_type=jnp.float32)


_fusion = pl.pallas_call(
    _fusion_body,
    grid=(NT,),
    in_specs=[
        pl.BlockSpec((R_TILE, D), lambda b: (b, 0)),
        pl.BlockSpec((D, K), lambda b: (0, 0)),
        pl.BlockSpec((1, K), lambda b: (0, 0)),
        pl.BlockSpec((D, K), lambda b: (0, 0)),
        pl.BlockSpec((1, K), lambda b: (0, 0)),
        pl.BlockSpec((K, D), lambda b: (0, 0)),
        pl.BlockSpec((K, D), lambda b: (0, 0)),
    ],
    out_specs=pl.BlockSpec((R_TILE, D), lambda b: (b, 0)),
    out_shape=jax.ShapeDtypeStruct((N_PAD, D), jnp.float32),
)


def _mean_body(a, b, c, d, out):
    out[...] = 0.25 * (a[...] + b[...] + c[...] + d[...])


_mean = pl.pallas_call(
    _mean_body,
    grid=(NT,),
    in_specs=[pl.BlockSpec((R_TILE, D), lambda b: (b, 0)) for _ in range(4)],
    out_specs=pl.BlockSpec((R_TILE, D), lambda b: (b, 0)),
    out_shape=jax.ShapeDtypeStruct((N_PAD, D), jnp.float32),
)


def kernel(edge_index, edge_weight, user_emb, item_emb, user_coll, item_coll, Wu, bu, Wi, bi):
    row = edge_index[0].astype(jnp.int32)
    col = edge_index[1].astype(jnp.int32)
    row = row + PADROWS * (row >= N_USERS).astype(jnp.int32)
    col = col + PADROWS * (col >= N_USERS).astype(jnp.int32)
    npad = E_PAD - E
    row = jnp.concatenate([row, jnp.full((npad,), N_PAD - 1, jnp.int32)])
    col = jnp.concatenate([col, jnp.zeros((npad,), jnp.int32)])
    w = jnp.concatenate([edge_weight, jnp.zeros((npad,), jnp.float32)])

    rows_g, cols_g, ws_g, bnd = _bucketize(row, col, w)
    bnd = bnd.reshape(96)

    emb_pad = jnp.concatenate(
        [user_emb, jnp.zeros((PADROWS, D), jnp.float32),
         item_emb, jnp.zeros((PADROWS, D), jnp.float32)], axis=0)

    x0 = _fusion(emb_pad, Wu, bu.reshape(1, K), Wi, bi.reshape(1, K),
                 user_coll, item_coll)
    x1 = _layer(x0, cols_g, rows_g, ws_g, bnd)
    x2 = _layer(x1, cols_g, rows_g, ws_g, bnd)
    x3 = _layer(x2, cols_g, rows_g, ws_g, bnd)
    fin = _mean(x0, x1, x2, x3)
    return fin[:N_USERS], fin[HALF:HALF + N_ITEMS]


# speedup vs baseline: 3.0324x; 1.0309x over previous
import functools

import jax
import jax.numpy as jnp
from jax import lax
from jax.experimental import pallas as pl
from jax.experimental.pallas import tpu as pltpu
from jax.experimental.pallas import tpu_sc as plsc

N_USERS = 25000
N_ITEMS = 25000
D = 64
E = 800000
K = 4
ALPHA_HALF = 0.5

NT = 32
R_TILE = 1568
HALF = 25088
N_PAD = 2 * HALF
PADROWS = HALF - N_USERS
CHUNK = 128
E_TILE = 25088
E_PAD = NT * E_TILE
E_SC = E_PAD // 2

_SC_MESH = plsc.VectorSubcoreMesh(core_axis_name="c", subcore_axis_name="s")
_IOTA16 = None


def _bucket_of(rowv):
    return ((rowv >> 5) * 669) >> 15


def _dup_stats(bv, iota):
    one = jnp.ones((16,), jnp.int32)
    zero = jnp.zeros((16,), jnp.int32)
    rankv = zero
    countv = zero
    for j in range(16):
        eq = bv == bv[j]
        countv = countv + jnp.where(eq, one, zero)
        rankv = rankv + jnp.where(jnp.logical_and(eq, iota > j), one, zero)
    return rankv, countv


@functools.partial(
    pl.kernel,
    mesh=_SC_MESH,
    compiler_params=pltpu.CompilerParams(use_tc_tiling_on_sc=False,
                                         needs_layout_passes=False),
    out_type=(
        jax.ShapeDtypeStruct((E_PAD,), jnp.int32),
        jax.ShapeDtypeStruct((E_PAD,), jnp.int32),
        jax.ShapeDtypeStruct((E_PAD,), jnp.float32),
        jax.ShapeDtypeStruct((2, 48), jnp.int32),
    ),
    scratch_types=[
        pltpu.VMEM((CHUNK,), jnp.int32),
        pltpu.VMEM((CHUNK,), jnp.int32),
        pltpu.VMEM((CHUNK,), jnp.float32),
        pltpu.VMEM((CHUNK,), jnp.int32),
        pltpu.VMEM((16, 32), jnp.int32),
        pltpu.VMEM((32,), jnp.int32),
        pltpu.VMEM((48,), jnp.int32),
        pltpu.VMEM((32,), jnp.int32),
        pltpu.VMEM_SHARED((16, 32), jnp.int32),
        pltpu.VMEM_SHARED((E_SC,), jnp.int32),
        pltpu.VMEM_SHARED((E_SC,), jnp.int32),
        pltpu.VMEM_SHARED((E_SC,), jnp.float32),
        pltpu.SemaphoreType.DMA,
        pltpu.SemaphoreType.DMA,
    ],
)
def _bucketize(row_hbm, col_hbm, w_hbm,
               rows_out, cols_out, ws_out, bnd_out,
               rowbuf, colbuf, wbuf, dstbuf, cntbuf, mycnt, bndbuf, offs,
               hist_sp, rows_sp, cols_sp, ws_sp, sem_a, sem_b):
    cid = lax.axis_index("c")
    sid = lax.axis_index("s")
    ebase = (cid * 16 + sid) * E_TILE
    iota = lax.iota(jnp.int32, 16)
    zero16 = jnp.zeros((16,), jnp.int32)

    mycnt[pl.ds(0, 16)] = zero16
    mycnt[pl.ds(16, 16)] = zero16

    def _hist_chunk(c, carry):
        pltpu.sync_copy(row_hbm.at[pl.ds(ebase + c * CHUNK, CHUNK)], rowbuf)

        def _grp(g, gc):
            bv = _bucket_of(rowbuf[pl.ds(g * 16, 16)])
            rankv, countv = _dup_stats(bv, iota)
            islast = rankv == countv - 1
            plsc.addupdate_scatter(mycnt, [bv], countv, mask=islast)
            return gc

        return lax.fori_loop(0, CHUNK // 16, _grp, carry)

    lax.fori_loop(0, E_TILE // CHUNK, _hist_chunk, 0)
    pltpu.sync_copy(mycnt, hist_sp.at[sid])
    plsc.subcore_barrier()

    pltpu.sync_copy(hist_sp, cntbuf)
    t0 = zero16
    t1 = zero16
    p0 = zero16
    p1 = zero16
    for t in range(16):
        r0 = cntbuf[t, pl.ds(0, 16)]
        r1 = cntbuf[t, pl.ds(16, 16)]
        m = t < sid
        p0 = p0 + jnp.where(m, r0, 0)
        p1 = p1 + jnp.where(m, r1, 0)
        t0 = t0 + r0
        t1 = t1 + r1
    run = jnp.int32(0)
    ex0 = zero16
    ex1 = zero16
    for b in range(16):
        ex0 = jnp.where(iota == b, run, ex0)
        run = run + t0[b]
    for b in range(16):
        ex1 = jnp.where(iota == b, run, ex1)
        run = run + t1[b]
    offs[pl.ds(0, 16)] = ex0 + p0
    offs[pl.ds(16, 16)] = ex1 + p1

    @pl.when(sid == 0)
    def _write_bnd():
        bndbuf[pl.ds(0, 16)] = ex0
        bndbuf[pl.ds(16, 16)] = ex1
        bndbuf[pl.ds(32, 16)] = jnp.where(iota == 0, E_SC, 0)
        pltpu.sync_copy(bndbuf, bnd_out.at[cid])

    def _scat_chunk(c, carry):
        eb = ebase + c * CHUNK
        d1 = pltpu.async_copy(row_hbm.at[pl.ds(eb, CHUNK)], rowbuf, sem_a)
        d2 = pltpu.async_copy(col_hbm.at[pl.ds(eb, CHUNK)], colbuf, sem_a)
        d3 = pltpu.async_copy(w_hbm.at[pl.ds(eb, CHUNK)], wbuf, sem_a)
        d1.wait()
        d2.wait()
        d3.wait()

        def _grp(g, gc):
            bv = _bucket_of(rowbuf[pl.ds(g * 16, 16)])
            rankv, countv = _dup_stats(bv, iota)
            islast = rankv == countv - 1
            dstv = plsc.load_gather(offs, [bv]) + rankv
            dstbuf[pl.ds(g * 16, 16)] = dstv
            plsc.addupdate_scatter(offs, [bv], countv, mask=islast)
            return gc

        lax.fori_loop(0, CHUNK // 16, _grp, 0)
        pltpu.sync_copy(rowbuf, rows_sp.at[dstbuf])
        pltpu.sync_copy(colbuf, cols_sp.at[dstbuf])
        pltpu.sync_copy(wbuf, ws_sp.at[dstbuf])
        return carry

    lax.fori_loop(0, E_TILE // CHUNK, _scat_chunk, 0)
    plsc.subcore_barrier()

    sl = E_SC // 16
    hb = cid * E_SC + sid * sl
    pltpu.sync_copy(rows_sp.at[pl.ds(sid * sl, sl)], rows_out.at[pl.ds(hb, sl)])
    pltpu.sync_copy(cols_sp.at[pl.ds(sid * sl, sl)], cols_out.at[pl.ds(hb, sl)])
    pltpu.sync_copy(ws_sp.at[pl.ds(sid * sl, sl)], ws_out.at[pl.ds(hb, sl)])


@functools.partial(
    pl.kernel,
    mesh=_SC_MESH,
    compiler_params=pltpu.CompilerParams(use_tc_tiling_on_sc=False,
                                         needs_layout_passes=False),
    out_type=jax.ShapeDtypeStruct((N_PAD, D), jnp.float32),
    scratch_types=[
        pltpu.VMEM((2, CHUNK), jnp.int32),
        pltpu.VMEM((2, CHUNK), jnp.int32),
        pltpu.VMEM((2, CHUNK), jnp.float32),
        pltpu.VMEM((2, CHUNK, D), jnp.float32),
        pltpu.VMEM((96,), jnp.int32),
        pltpu.VMEM((R_TILE, D), jnp.float32),
        pltpu.SemaphoreType.DMA,
        pltpu.SemaphoreType.DMA,
        pltpu.SemaphoreType.DMA,
    ],
)
def _layer(x_hbm, cs_hbm, rs_hbm, ws_hbm, bnd_hbm, out_hbm,
           colbuf, rowbuf, wbuf, gbuf, bndbuf, acc, sem_e, sem_g, sem_g2):
    wid = lax.axis_index("s") * 2 + lax.axis_index("c")
    base = wid * R_TILE
    pltpu.sync_copy(bnd_hbm, bndbuf)

    zero16 = jnp.zeros((16,), jnp.float32)

    def _zero_row(r, carry):
        for d4 in range(4):
            acc[r, pl.ds(d4 * 16, 16)] = zero16
        return carry

    lax.fori_loop(0, R_TILE, _zero_row, 0)

    def _issue_edges(c, p):
        eb = c * CHUNK
        pltpu.async_copy(cs_hbm.at[pl.ds(eb, CHUNK)], colbuf.at[p], sem_e)
        pltpu.async_copy(rs_hbm.at[pl.ds(eb, CHUNK)], rowbuf.at[p], sem_e)
        pltpu.async_copy(ws_hbm.at[pl.ds(eb, CHUNK)], wbuf.at[p], sem_e)

    def _drain_edges(c, p):
        eb = c * CHUNK
        pltpu.make_async_copy(cs_hbm.at[pl.ds(eb, CHUNK)], colbuf.at[p], sem_e).wait()
        pltpu.make_async_copy(rs_hbm.at[pl.ds(eb, CHUNK)], rowbuf.at[p], sem_e).wait()
        pltpu.make_async_copy(ws_hbm.at[pl.ds(eb, CHUNK)], wbuf.at[p], sem_e).wait()

    H = CHUNK // 2

    def _issue_gather(p):
        pltpu.async_copy(x_hbm.at[colbuf.at[p, pl.ds(0, H)]],
                         gbuf.at[p, pl.ds(0, H)], sem_g)
        pltpu.async_copy(x_hbm.at[colbuf.at[p, pl.ds(H, H)]],
                         gbuf.at[p, pl.ds(H, H)], sem_g2)

    def _drain_gather(p):
        pltpu.make_async_copy(x_hbm.at[colbuf.at[p, pl.ds(0, H)]],
                              gbuf.at[p, pl.ds(0, H)], sem_g).wait()
        pltpu.make_async_copy(x_hbm.at[colbuf.at[p, pl.ds(H, H)]],
                              gbuf.at[p, pl.ds(H, H)], sem_g2).wait()

    for seg in range(2):
        sc_base = seg * E_SC
        bv = bndbuf[pl.ds(seg * 48 + wid, 16)]
        e0 = bv[0] + sc_base
        e1 = bv[1] + sc_base
        c0 = e0 // CHUNK
        c1 = (e1 + (CHUNK - 1)) // CHUNK

        @pl.when(c0 < c1)
        def _prologue():
            _issue_edges(c0, 0)
            _drain_edges(c0, 0)
            _issue_gather(0)

        def _chunk(c, carry):
            p = lax.rem(c - c0, 2)
            q = 1 - p
            _drain_gather(p)

            @pl.when(c + 1 < c1)
            def _prefetch_edges():
                _issue_edges(c + 1, q)

            def _group(g, gcarry):
                rowv = rowbuf[p, pl.ds(g * 16, 16)] - base
                validv = jnp.logical_and(rowv >= 0, rowv < R_TILE)
                wv = jnp.where(validv, wbuf[p, pl.ds(g * 16, 16)], 0.0)
                rlv = jnp.where(validv, rowv, 0)
                ws = [wv[k] for k in range(16)]
                rls = [rlv[k] for k in range(16)]
                for k in range(16):
                    e = g * 16 + k
                    w = ws[k]
                    rl = rls[k]
                    for d4 in range(4):
                        gvec = gbuf[p, e, pl.ds(d4 * 16, 16)]
                        plsc.addupdate(acc.at[rl, pl.ds(d4 * 16, 16)], gvec * w)
                return gcarry

            lax.fori_loop(0, CHUNK // 16, _group, 0)

            @pl.when(c + 1 < c1)
            def _prefetch_gather():
                _drain_edges(c + 1, q)
                _issue_gather(q)

            return carry

        lax.fori_loop(c0, c1, _chunk, 0)

    pltpu.sync_copy(acc, out_hbm.at[pl.ds(base, R_TILE)])


def _fusion_body(emb, wu, bu, wi, bi, uc, ic, out):
    b = pl.program_id(0)
    x = emb[...]
    isu = b < 16
    W = jnp.where(isu, wu[...], wi[...])
    bb = jnp.where(isu, bu[...], bi[...])
    C = jnp.where(isu, uc[...], ic[...])
    logits = jnp.dot(x, W, preferred_element_type=jnp.float32) + bb
    m = jnp.max(logits, axis=-1, keepdims=True)
    ex = jnp.exp(logits - m)
    attn = ex / jnp.sum(ex, axis=-1, keepdims=True)
    out[...] = x + ALPHA_HALF * jnp.dot(attn, C, preferred_element_type=jnp.float32)


_fusion = pl.pallas_call(
    _fusion_body,
    grid=(NT,),
    in_specs=[
        pl.BlockSpec((R_TILE, D), lambda b: (b, 0)),
        pl.BlockSpec((D, K), lambda b: (0, 0)),
        pl.BlockSpec((1, K), lambda b: (0, 0)),
        pl.BlockSpec((D, K), lambda b: (0, 0)),
        pl.BlockSpec((1, K), lambda b: (0, 0)),
        pl.BlockSpec((K, D), lambda b: (0, 0)),
        pl.BlockSpec((K, D), lambda b: (0, 0)),
    ],
    out_specs=pl.BlockSpec((R_TILE, D), lambda b: (b, 0)),
    out_shape=jax.ShapeDtypeStruct((N_PAD, D), jnp.float32),
)


def _mean_body(a, b, c, d, out):
    out[...] = 0.25 * (a[...] + b[...] + c[...] + d[...])


_mean = pl.pallas_call(
    _mean_body,
    grid=(NT,),
    in_specs=[pl.BlockSpec((R_TILE, D), lambda b: (b, 0)) for _ in range(4)],
    out_specs=pl.BlockSpec((R_TILE, D), lambda b: (b, 0)),
    out_shape=jax.ShapeDtypeStruct((N_PAD, D), jnp.float32),
)


def kernel(edge_index, edge_weight, user_emb, item_emb, user_coll, item_coll, Wu, bu, Wi, bi):
    row = edge_index[0].astype(jnp.int32)
    col = edge_index[1].astype(jnp.int32)
    row = row + PADROWS * (row >= N_USERS).astype(jnp.int32)
    col = col + PADROWS * (col >= N_USERS).astype(jnp.int32)
    npad = E_PAD - E
    row = jnp.concatenate([row, jnp.full((npad,), N_PAD - 1, jnp.int32)])
    col = jnp.concatenate([col, jnp.zeros((npad,), jnp.int32)])
    w = jnp.concatenate([edge_weight, jnp.zeros((npad,), jnp.float32)])

    rows_g, cols_g, ws_g, bnd = _bucketize(row, col, w)
    bnd = bnd.reshape(96)

    emb_pad = jnp.concatenate(
        [user_emb, jnp.zeros((PADROWS, D), jnp.float32),
         item_emb, jnp.zeros((PADROWS, D), jnp.float32)], axis=0)

    x0 = _fusion(emb_pad, Wu, bu.reshape(1, K), Wi, bi.reshape(1, K),
                 user_coll, item_coll)
    x1 = _layer(x0, cols_g, rows_g, ws_g, bnd)
    x2 = _layer(x1, cols_g, rows_g, ws_g, bnd)
    x3 = _layer(x2, cols_g, rows_g, ws_g, bnd)
    fin = _mean(x0, x1, x2, x3)
    return fin[:N_USERS], fin[HALF:HALF + N_ITEMS]


# speedup vs baseline: 3.1655x; 1.0439x over previous
import functools

import jax
import jax.numpy as jnp
from jax import lax
from jax.experimental import pallas as pl
from jax.experimental.pallas import tpu as pltpu
from jax.experimental.pallas import tpu_sc as plsc

N_USERS = 25000
N_ITEMS = 25000
D = 64
E = 800000
K = 4
ALPHA_HALF = 0.5

NT = 32
R_TILE = 1568
HALF = 25088
N_PAD = 2 * HALF
PADROWS = HALF - N_USERS
CHUNK = 128
E_TILE = 25088
E_PAD = NT * E_TILE
E_SC = E_PAD // 2

_SC_MESH = plsc.VectorSubcoreMesh(core_axis_name="c", subcore_axis_name="s")
_IOTA16 = None


def _bucket_of(rowv):
    return ((rowv >> 5) * 669) >> 15


def _dup_stats(bv, iota):
    one = jnp.ones((16,), jnp.int32)
    zero = jnp.zeros((16,), jnp.int32)
    rankv = zero
    countv = zero
    for j in range(16):
        eq = bv == bv[j]
        countv = countv + jnp.where(eq, one, zero)
        rankv = rankv + jnp.where(jnp.logical_and(eq, iota > j), one, zero)
    return rankv, countv


@functools.partial(
    pl.kernel,
    mesh=_SC_MESH,
    compiler_params=pltpu.CompilerParams(use_tc_tiling_on_sc=False,
                                         needs_layout_passes=False),
    out_type=(
        jax.ShapeDtypeStruct((E_PAD,), jnp.int32),
        jax.ShapeDtypeStruct((E_PAD,), jnp.int32),
        jax.ShapeDtypeStruct((E_PAD,), jnp.float32),
        jax.ShapeDtypeStruct((2, 48), jnp.int32),
    ),
    scratch_types=[
        pltpu.VMEM((2, CHUNK), jnp.int32),
        pltpu.VMEM((2, CHUNK), jnp.int32),
        pltpu.VMEM((2, CHUNK), jnp.float32),
        pltpu.VMEM((CHUNK,), jnp.int32),
        pltpu.VMEM((16, 32), jnp.int32),
        pltpu.VMEM((32,), jnp.int32),
        pltpu.VMEM((48,), jnp.int32),
        pltpu.VMEM((32,), jnp.int32),
        pltpu.VMEM_SHARED((16, 32), jnp.int32),
        pltpu.VMEM_SHARED((E_SC,), jnp.int32),
        pltpu.VMEM_SHARED((E_SC,), jnp.int32),
        pltpu.VMEM_SHARED((E_SC,), jnp.float32),
        pltpu.SemaphoreType.DMA,
        pltpu.SemaphoreType.DMA,
    ],
)
def _bucketize(row_hbm, col_hbm, w_hbm,
               rows_out, cols_out, ws_out, bnd_out,
               rowbuf, colbuf, wbuf, dstbuf, cntbuf, mycnt, bndbuf, offs,
               hist_sp, rows_sp, cols_sp, ws_sp, sem_a, sem_b):
    cid = lax.axis_index("c")
    sid = lax.axis_index("s")
    ebase = (cid * 16 + sid) * E_TILE
    iota = lax.iota(jnp.int32, 16)
    zero16 = jnp.zeros((16,), jnp.int32)

    mycnt[pl.ds(0, 16)] = zero16
    mycnt[pl.ds(16, 16)] = zero16

    NCH = E_TILE // CHUNK

    def _issue_row(c, p):
        pltpu.async_copy(row_hbm.at[pl.ds(ebase + c * CHUNK, CHUNK)],
                         rowbuf.at[p], sem_a)

    def _drain_row(c, p):
        pltpu.make_async_copy(row_hbm.at[pl.ds(ebase + c * CHUNK, CHUNK)],
                              rowbuf.at[p], sem_a).wait()

    _issue_row(0, 0)

    def _hist_chunk(c, carry):
        p = lax.rem(c, 2)
        _drain_row(c, p)

        @pl.when(c + 1 < NCH)
        def _pf():
            _issue_row(c + 1, 1 - p)

        def _grp(g, gc):
            bv = _bucket_of(rowbuf[p, pl.ds(g * 16, 16)])
            rankv, countv = _dup_stats(bv, iota)
            islast = rankv == countv - 1
            plsc.addupdate_scatter(mycnt, [bv], countv, mask=islast)
            return gc

        return lax.fori_loop(0, CHUNK // 16, _grp, carry)

    lax.fori_loop(0, NCH, _hist_chunk, 0)
    pltpu.sync_copy(mycnt, hist_sp.at[sid])
    plsc.subcore_barrier()

    pltpu.sync_copy(hist_sp, cntbuf)
    t0 = zero16
    t1 = zero16
    p0 = zero16
    p1 = zero16
    for t in range(16):
        r0 = cntbuf[t, pl.ds(0, 16)]
        r1 = cntbuf[t, pl.ds(16, 16)]
        m = t < sid
        p0 = p0 + jnp.where(m, r0, 0)
        p1 = p1 + jnp.where(m, r1, 0)
        t0 = t0 + r0
        t1 = t1 + r1
    run = jnp.int32(0)
    ex0 = zero16
    ex1 = zero16
    for b in range(16):
        ex0 = jnp.where(iota == b, run, ex0)
        run = run + t0[b]
    for b in range(16):
        ex1 = jnp.where(iota == b, run, ex1)
        run = run + t1[b]
    offs[pl.ds(0, 16)] = ex0 + p0
    offs[pl.ds(16, 16)] = ex1 + p1

    @pl.when(sid == 0)
    def _write_bnd():
        bndbuf[pl.ds(0, 16)] = ex0
        bndbuf[pl.ds(16, 16)] = ex1
        bndbuf[pl.ds(32, 16)] = jnp.where(iota == 0, E_SC, 0)
        pltpu.sync_copy(bndbuf, bnd_out.at[cid])

    def _issue_edges(c, p):
        eb = ebase + c * CHUNK
        pltpu.async_copy(row_hbm.at[pl.ds(eb, CHUNK)], rowbuf.at[p], sem_a)
        pltpu.async_copy(col_hbm.at[pl.ds(eb, CHUNK)], colbuf.at[p], sem_a)
        pltpu.async_copy(w_hbm.at[pl.ds(eb, CHUNK)], wbuf.at[p], sem_a)

    def _drain_edges(c, p):
        eb = ebase + c * CHUNK
        pltpu.make_async_copy(row_hbm.at[pl.ds(eb, CHUNK)], rowbuf.at[p], sem_a).wait()
        pltpu.make_async_copy(col_hbm.at[pl.ds(eb, CHUNK)], colbuf.at[p], sem_a).wait()
        pltpu.make_async_copy(w_hbm.at[pl.ds(eb, CHUNK)], wbuf.at[p], sem_a).wait()

    _issue_edges(0, 0)

    def _scat_chunk(c, carry):
        p = lax.rem(c, 2)
        _drain_edges(c, p)

        @pl.when(c + 1 < NCH)
        def _pf():
            _issue_edges(c + 1, 1 - p)

        def _grp(g, gc):
            bv = _bucket_of(rowbuf[p, pl.ds(g * 16, 16)])
            rankv, countv = _dup_stats(bv, iota)
            islast = rankv == countv - 1
            dstv = plsc.load_gather(offs, [bv]) + rankv
            dstbuf[pl.ds(g * 16, 16)] = dstv
            plsc.addupdate_scatter(offs, [bv], countv, mask=islast)
            return gc

        lax.fori_loop(0, CHUNK // 16, _grp, 0)
        pltpu.sync_copy(rowbuf.at[p], rows_sp.at[dstbuf])
        pltpu.sync_copy(colbuf.at[p], cols_sp.at[dstbuf])
        pltpu.sync_copy(wbuf.at[p], ws_sp.at[dstbuf])
        return carry

    lax.fori_loop(0, NCH, _scat_chunk, 0)
    plsc.subcore_barrier()

    sl = E_SC // 16
    hb = cid * E_SC + sid * sl
    pltpu.sync_copy(rows_sp.at[pl.ds(sid * sl, sl)], rows_out.at[pl.ds(hb, sl)])
    pltpu.sync_copy(cols_sp.at[pl.ds(sid * sl, sl)], cols_out.at[pl.ds(hb, sl)])
    pltpu.sync_copy(ws_sp.at[pl.ds(sid * sl, sl)], ws_out.at[pl.ds(hb, sl)])


@functools.partial(
    pl.kernel,
    mesh=_SC_MESH,
    compiler_params=pltpu.CompilerParams(use_tc_tiling_on_sc=False,
                                         needs_layout_passes=False),
    out_type=jax.ShapeDtypeStruct((N_PAD, D), jnp.float32),
    scratch_types=[
        pltpu.VMEM((2, CHUNK), jnp.int32),
        pltpu.VMEM((2, CHUNK), jnp.int32),
        pltpu.VMEM((2, CHUNK), jnp.float32),
        pltpu.VMEM((2, CHUNK, D), jnp.float32),
        pltpu.VMEM((96,), jnp.int32),
        pltpu.VMEM((R_TILE, D), jnp.float32),
        pltpu.SemaphoreType.DMA,
        pltpu.SemaphoreType.DMA,
        pltpu.SemaphoreType.DMA,
    ],
)
def _layer(x_hbm, cs_hbm, rs_hbm, ws_hbm, bnd_hbm, out_hbm,
           colbuf, rowbuf, wbuf, gbuf, bndbuf, acc, sem_e, sem_g, sem_g2):
    wid = lax.axis_index("s") * 2 + lax.axis_index("c")
    base = wid * R_TILE
    pltpu.sync_copy(bnd_hbm, bndbuf)

    zero16 = jnp.zeros((16,), jnp.float32)

    def _zero_row(r, carry):
        for d4 in range(4):
            acc[r, pl.ds(d4 * 16, 16)] = zero16
        return carry

    lax.fori_loop(0, R_TILE, _zero_row, 0)

    def _issue_edges(c, p):
        eb = c * CHUNK
        pltpu.async_copy(cs_hbm.at[pl.ds(eb, CHUNK)], colbuf.at[p], sem_e)
        pltpu.async_copy(rs_hbm.at[pl.ds(eb, CHUNK)], rowbuf.at[p], sem_e)
        pltpu.async_copy(ws_hbm.at[pl.ds(eb, CHUNK)], wbuf.at[p], sem_e)

    def _drain_edges(c, p):
        eb = c * CHUNK
        pltpu.make_async_copy(cs_hbm.at[pl.ds(eb, CHUNK)], colbuf.at[p], sem_e).wait()
        pltpu.make_async_copy(rs_hbm.at[pl.ds(eb, CHUNK)], rowbuf.at[p], sem_e).wait()
        pltpu.make_async_copy(ws_hbm.at[pl.ds(eb, CHUNK)], wbuf.at[p], sem_e).wait()

    H = CHUNK // 2

    def _issue_gather(p):
        pltpu.async_copy(x_hbm.at[colbuf.at[p, pl.ds(0, H)]],
                         gbuf.at[p, pl.ds(0, H)], sem_g)
        pltpu.async_copy(x_hbm.at[colbuf.at[p, pl.ds(H, H)]],
                         gbuf.at[p, pl.ds(H, H)], sem_g2)

    def _drain_gather(p):
        pltpu.make_async_copy(x_hbm.at[colbuf.at[p, pl.ds(0, H)]],
                              gbuf.at[p, pl.ds(0, H)], sem_g).wait()
        pltpu.make_async_copy(x_hbm.at[colbuf.at[p, pl.ds(H, H)]],
                              gbuf.at[p, pl.ds(H, H)], sem_g2).wait()

    for seg in range(2):
        sc_base = seg * E_SC
        bv = bndbuf[pl.ds(seg * 48 + wid, 16)]
        e0 = bv[0] + sc_base
        e1 = bv[1] + sc_base
        c0 = e0 // CHUNK
        c1 = (e1 + (CHUNK - 1)) // CHUNK

        @pl.when(c0 < c1)
        def _prologue():
            _issue_edges(c0, 0)
            _drain_edges(c0, 0)
            _issue_gather(0)

        def _chunk(c, carry):
            p = lax.rem(c - c0, 2)
            q = 1 - p
            _drain_gather(p)

            @pl.when(c + 1 < c1)
            def _prefetch_edges():
                _issue_edges(c + 1, q)

            def _group(g, gcarry):
                rowv = rowbuf[p, pl.ds(g * 16, 16)] - base
                validv = jnp.logical_and(rowv >= 0, rowv < R_TILE)
                wv = jnp.where(validv, wbuf[p, pl.ds(g * 16, 16)], 0.0)
                rlv = jnp.where(validv, rowv, 0)
                ws = [wv[k] for k in range(16)]
                rls = [rlv[k] for k in range(16)]
                for k in range(16):
                    e = g * 16 + k
                    w = ws[k]
                    rl = rls[k]
                    for d4 in range(4):
                        gvec = gbuf[p, e, pl.ds(d4 * 16, 16)]
                        plsc.addupdate(acc.at[rl, pl.ds(d4 * 16, 16)], gvec * w)
                return gcarry

            lax.fori_loop(0, CHUNK // 16, _group, 0)

            @pl.when(c + 1 < c1)
            def _prefetch_gather():
                _drain_edges(c + 1, q)
                _issue_gather(q)

            return carry

        lax.fori_loop(c0, c1, _chunk, 0)

    pltpu.sync_copy(acc, out_hbm.at[pl.ds(base, R_TILE)])


def _fusion_body(emb, wu, bu, wi, bi, uc, ic, out):
    b = pl.program_id(0)
    x = emb[...]
    isu = b < 16
    W = jnp.where(isu, wu[...], wi[...])
    bb = jnp.where(isu, bu[...], bi[...])
    C = jnp.where(isu, uc[...], ic[...])
    logits = jnp.dot(x, W, preferred_element_type=jnp.float32) + bb
    m = jnp.max(logits, axis=-1, keepdims=True)
    ex = jnp.exp(logits - m)
    attn = ex / jnp.sum(ex, axis=-1, keepdims=True)
    out[...] = x + ALPHA_HALF * jnp.dot(attn, C, preferred_element_type=jnp.float32)


_fusion = pl.pallas_call(
    _fusion_body,
    grid=(NT,),
    in_specs=[
        pl.BlockSpec((R_TILE, D), lambda b: (b, 0)),
        pl.BlockSpec((D, K), lambda b: (0, 0)),
        pl.BlockSpec((1, K), lambda b: (0, 0)),
        pl.BlockSpec((D, K), lambda b: (0, 0)),
        pl.BlockSpec((1, K), lambda b: (0, 0)),
        pl.BlockSpec((K, D), lambda b: (0, 0)),
        pl.BlockSpec((K, D), lambda b: (0, 0)),
    ],
    out_specs=pl.BlockSpec((R_TILE, D), lambda b: (b, 0)),
    out_shape=jax.ShapeDtypeStruct((N_PAD, D), jnp.float32),
)


def _mean_body(a, b, c, d, out):
    out[...] = 0.25 * (a[...] + b[...] + c[...] + d[...])


_mean = pl.pallas_call(
    _mean_body,
    grid=(NT,),
    in_specs=[pl.BlockSpec((R_TILE, D), lambda b: (b, 0)) for _ in range(4)],
    out_specs=pl.BlockSpec((R_TILE, D), lambda b: (b, 0)),
    out_shape=jax.ShapeDtypeStruct((N_PAD, D), jnp.float32),
)


def kernel(edge_index, edge_weight, user_emb, item_emb, user_coll, item_coll, Wu, bu, Wi, bi):
    row = edge_index[0].astype(jnp.int32)
    col = edge_index[1].astype(jnp.int32)
    row = row + PADROWS * (row >= N_USERS).astype(jnp.int32)
    col = col + PADROWS * (col >= N_USERS).astype(jnp.int32)
    npad = E_PAD - E
    row = jnp.concatenate([row, jnp.full((npad,), N_PAD - 1, jnp.int32)])
    col = jnp.concatenate([col, jnp.zeros((npad,), jnp.int32)])
    w = jnp.concatenate([edge_weight, jnp.zeros((npad,), jnp.float32)])

    rows_g, cols_g, ws_g, bnd = _bucketize(row, col, w)
    bnd = bnd.reshape(96)

    emb_pad = jnp.concatenate(
        [user_emb, jnp.zeros((PADROWS, D), jnp.float32),
         item_emb, jnp.zeros((PADROWS, D), jnp.float32)], axis=0)

    x0 = _fusion(emb_pad, Wu, bu.reshape(1, K), Wi, bi.reshape(1, K),
                 user_coll, item_coll)
    x1 = _layer(x0, cols_g, rows_g, ws_g, bnd)
    x2 = _layer(x1, cols_g, rows_g, ws_g, bnd)
    x3 = _layer(x2, cols_g, rows_g, ws_g, bnd)
    fin = _mean(x0, x1, x2, x3)
    return fin[:N_USERS], fin[HALF:HALF + N_ITEMS]
